# baseline (speedup 1.0000x reference)
import functools

import jax
import jax.numpy as jnp
from jax import lax
from jax.experimental import pallas as pl
from jax.experimental.pallas import tpu as pltpu
from jax.experimental.pallas import tpu_sc as plsc

_V = 1000
_H = 128
_B = 4096 * 20
_NC = 2
_NS = 16
_NW = _NC * _NS
_BPW = _B // _NW
_CH = 64


def _mm_body(emb_ref, w_ref, b_ref, out_ref):
    out_ref[...] = lax.dot_general(
        emb_ref[...], w_ref[...], (((1,), (1,)), ((), ())),
        preferred_element_type=jnp.float32) + b_ref[...]


def _make_table(emb_table, lin_w, lin_b2d):
    return pl.pallas_call(
        _mm_body,
        out_shape=jax.ShapeDtypeStruct((_V, _V), jnp.float32),
    )(emb_table, lin_w, lin_b2d)


@functools.lru_cache(maxsize=1)
def _make_gather():
    mesh = plsc.VectorSubcoreMesh(core_axis_name="c", subcore_axis_name="s")

    @functools.partial(
        pl.kernel,
        mesh=mesh,
        out_type=jax.ShapeDtypeStruct((_B, _V), jnp.float32),
        scratch_types=[
            pltpu.VMEM((_BPW,), jnp.int32),
            pltpu.VMEM((_CH, _V), jnp.float32),
            pltpu.SemaphoreType.DMA,
        ],
        compiler_params=pltpu.CompilerParams(use_tc_tiling_on_sc=False),
    )
    def _gather_rows(m_hbm, idx_hbm, out_hbm, idx_v, rows_v, sem):
        wid = lax.axis_index("s") * _NC + lax.axis_index("c")
        base = wid * _BPW
        pltpu.sync_copy(idx_hbm.at[pl.ds(base, _BPW)], idx_v)

        def body(i, carry):
            off = pl.multiple_of(i * _CH, 8)
            pltpu.async_copy(m_hbm.at[idx_v.at[pl.ds(off, _CH)]], rows_v, sem).wait()
            pltpu.sync_copy(rows_v, out_hbm.at[pl.ds(base + off, _CH)])
            return carry

        lax.fori_loop(0, _BPW // _CH, body, 0)

    return _gather_rows


def kernel(input_ids, emb_table, lin_w, lin_b):
    table = _make_table(emb_table, lin_w, lin_b.reshape(1, _V))
    ids = input_ids.reshape(-1).astype(jnp.int32)
    out = _make_gather()(table, ids)
    return out.reshape(input_ids.shape + (_V,))


# speedup vs baseline: 1.0092x; 1.0092x over previous
import functools

import jax
import jax.numpy as jnp
from jax import lax
from jax.experimental import pallas as pl
from jax.experimental.pallas import tpu as pltpu
from jax.experimental.pallas import tpu_sc as plsc

_V = 1000
_H = 128
_B = 4096 * 20
_NC = 2
_NS = 16
_NW = _NC * _NS
_BPW = _B // _NW
_CH = 40


def _mm_body(emb_ref, w_ref, b_ref, out_ref):
    out_ref[...] = lax.dot_general(
        emb_ref[...], w_ref[...], (((1,), (1,)), ((), ())),
        preferred_element_type=jnp.float32) + b_ref[...]


def _make_table(emb_table, lin_w, lin_b2d):
    return pl.pallas_call(
        _mm_body,
        out_shape=jax.ShapeDtypeStruct((_V, _V), jnp.float32),
    )(emb_table, lin_w, lin_b2d)


@functools.lru_cache(maxsize=1)
def _make_gather():
    mesh = plsc.VectorSubcoreMesh(core_axis_name="c", subcore_axis_name="s")

    @functools.partial(
        pl.kernel,
        mesh=mesh,
        out_type=jax.ShapeDtypeStruct((_B, _V), jnp.float32),
        scratch_types=[
            pltpu.VMEM((_BPW,), jnp.int32),
            pltpu.VMEM((_CH, _V), jnp.float32),
            pltpu.VMEM((_CH, _V), jnp.float32),
            pltpu.SemaphoreType.DMA,
            pltpu.SemaphoreType.DMA,
            pltpu.SemaphoreType.DMA,
            pltpu.SemaphoreType.DMA,
        ],
        compiler_params=pltpu.CompilerParams(use_tc_tiling_on_sc=False),
    )
    def _gather_rows(m_hbm, idx_hbm, out_hbm, idx_v, rows_a, rows_b,
                     sem_ga, sem_gb, sem_sa, sem_sb):
        wid = lax.axis_index("s") * _NC + lax.axis_index("c")
        base = wid * _BPW
        pltpu.sync_copy(idx_hbm.at[pl.ds(base, _BPW)], idx_v)

        def _gather(chunk, buf, sem):
            off = pl.multiple_of(chunk * _CH, 8)
            return pltpu.async_copy(m_hbm.at[idx_v.at[pl.ds(off, _CH)]], buf, sem)

        def _store(chunk, buf, sem):
            off = pl.multiple_of(chunk * _CH, 8)
            return pltpu.async_copy(buf, out_hbm.at[pl.ds(base + off, _CH)], sem)

        def _wait_store(buf, sem):
            pltpu.make_async_copy(buf, out_hbm.at[pl.ds(base, _CH)], sem).wait()

        def body(j, carry):
            i0 = 2 * j
            ga = _gather(i0, rows_a, sem_ga)

            @pl.when(j > 0)
            def _():
                _wait_store(rows_b, sem_sb)

            gb = _gather(i0 + 1, rows_b, sem_gb)
            ga.wait()
            _store(i0, rows_a, sem_sa)
            gb.wait()
            _store(i0 + 1, rows_b, sem_sb)

            @pl.when(j + 1 < _BPW // (2 * _CH))
            def _():
                _wait_store(rows_a, sem_sa)

            return carry

        lax.fori_loop(0, _BPW // (2 * _CH), body, 0)
        _wait_store(rows_a, sem_sa)
        _wait_store(rows_b, sem_sb)

    return _gather_rows


def kernel(input_ids, emb_table, lin_w, lin_b):
    table = _make_table(emb_table, lin_w, lin_b.reshape(1, _V))
    ids = input_ids.reshape(-1).astype(jnp.int32)
    out = _make_gather()(table, ids)
    return out.reshape(input_ids.shape + (_V,))
